# initial kernel scaffold (unmeasured)
import jax
import jax.numpy as jnp
from jax import lax
from jax.experimental import pallas as pl
from jax.experimental.pallas import tpu as pltpu

N_DEV = 32
M_PER = 128
K = 4096
N_PER = 256


def kernel(x, w_mat):
    def body(x_ref, w_hbm, out_ref, wbuf, ybuf, copy_sems, send_sems, recv_sems):
        me = lax.axis_index("i")

        def w_col(s):
            return lax.rem(me + s, N_DEV) * N_PER

        def w_copy(s, slot):
            return pltpu.make_async_copy(
                w_hbm.at[:, pl.ds(w_col(s), N_PER)],
                wbuf.at[slot],
                copy_sems.at[slot],
            )

        w_copy(0, 0).start()

        for s in range(N_DEV):
            slot = s % 2
            w_copy(s, slot).wait()
            if s + 1 < N_DEV:
                w_copy(s + 1, (s + 1) % 2).start()

            chunk = jnp.dot(
                x_ref[...], wbuf[slot], preferred_element_type=jnp.float32
            )
            if s == 0:
                out_ref[pl.ds(me * M_PER, M_PER), :] = chunk
            else:
                ybuf[s, :, :] = chunk
                target = lax.rem(me + s, N_DEV)
                rdma = pltpu.make_async_remote_copy(
                    src_ref=ybuf.at[s],
                    dst_ref=out_ref.at[pl.ds(me * M_PER, M_PER), :],
                    send_sem=send_sems.at[s],
                    recv_sem=recv_sems.at[s],
                    device_id=(target,),
                    device_id_type=pl.DeviceIdType.MESH,
                )
                rdma.start()
                rdma.wait()

    out_shape = jax.ShapeDtypeStruct((N_DEV * M_PER, N_PER), jnp.float32)
    return pl.pallas_call(
        body,
        out_shape=out_shape,
        in_specs=[
            pl.BlockSpec(memory_space=pltpu.VMEM),
            pl.BlockSpec(memory_space=pltpu.ANY),
        ],
        out_specs=pl.BlockSpec(memory_space=pltpu.VMEM),
        scratch_shapes=[
            pltpu.VMEM((2, K, N_PER), jnp.float32),
            pltpu.VMEM((N_DEV, M_PER, N_PER), jnp.float32),
            pltpu.SemaphoreType.DMA((2,)),
            pltpu.SemaphoreType.DMA((N_DEV,)),
            pltpu.SemaphoreType.DMA((N_DEV,)),
        ],
        compiler_params=pltpu.CompilerParams(collective_id=0),
    )(x, w_mat)


# baseline (device time: 185589 ns/iter reference)
import jax
import jax.numpy as jnp
from jax import lax
from jax.experimental import pallas as pl
from jax.experimental.pallas import tpu as pltpu

N_DEV = 32
M_PER = 128
K = 4096
N_PER = 256


def kernel(x, w_mat):
    def body(x_ref, w_hbm, out_ref, wbuf, ybuf, copy_sems, send_sems, recv_sems):
        me = lax.axis_index("i")

        def w_col(s):
            return lax.rem(me + s, N_DEV) * N_PER

        def w_copy(s, slot):
            return pltpu.make_async_copy(
                w_hbm.at[:, pl.ds(w_col(s), N_PER)],
                wbuf.at[slot],
                copy_sems.at[slot],
            )

        w_copy(0, 0).start()

        for s in range(N_DEV):
            slot = s % 2
            w_copy(s, slot).wait()
            if s + 1 < N_DEV:
                w_copy(s + 1, (s + 1) % 2).start()

            chunk = jnp.dot(
                x_ref[...], wbuf[slot], preferred_element_type=jnp.float32
            )
            if s == 0:
                out_ref[pl.ds(me * M_PER, M_PER), :] = chunk
            else:
                ybuf[s, :, :] = chunk
                target = lax.rem(me + s, N_DEV)
                rdma = pltpu.make_async_remote_copy(
                    src_ref=ybuf.at[s],
                    dst_ref=out_ref.at[pl.ds(me * M_PER, M_PER), :],
                    send_sem=send_sems.at[s],
                    recv_sem=recv_sems.at[s],
                    device_id=(target,),
                    device_id_type=pl.DeviceIdType.MESH,
                )
                rdma.start()
                rdma.wait()

    out_shape = jax.ShapeDtypeStruct((N_DEV * M_PER, N_PER), jnp.float32)
    return pl.pallas_call(
        body,
        out_shape=out_shape,
        in_specs=[
            pl.BlockSpec(memory_space=pltpu.VMEM),
            pl.BlockSpec(memory_space=pltpu.MemorySpace.HBM),
        ],
        out_specs=pl.BlockSpec(memory_space=pltpu.VMEM),
        scratch_shapes=[
            pltpu.VMEM((2, K, N_PER), jnp.float32),
            pltpu.VMEM((N_DEV, M_PER, N_PER), jnp.float32),
            pltpu.SemaphoreType.DMA((2,)),
            pltpu.SemaphoreType.DMA((N_DEV,)),
            pltpu.SemaphoreType.DMA((N_DEV,)),
        ],
    )(x, w_mat)
